# Initial kernel scaffold; baseline (speedup 1.0000x reference)
import jax
import jax.numpy as jnp
from jax.experimental import pallas as pl


def kernel(x, adj, current_samples):
    raise NotImplementedError("write your pallas kernel here")


# speedup vs baseline: 2.2700x; 2.2700x over previous
import functools

import jax
import jax.numpy as jnp
from jax import lax
from jax.experimental import pallas as pl
from jax.experimental.pallas import tpu as pltpu
from jax.experimental.pallas import tpu_sc as plsc

_N = 2048
_D = 256
_K = 64

_NUM_WORKERS = 32
_ROWS_PER_WORKER = _N // _NUM_WORKERS
_BATCH = 8
_LANES = 16

_MASKVAL = -1e30


def _sc_build_s_body(cs_hbm, s_hbm, cs_v, buf, sem):
    wid = lax.axis_index("s") * 2 + lax.axis_index("c")
    row_base = wid * _ROWS_PER_WORKER

    pltpu.sync_copy(cs_hbm.at[pl.ds(row_base * _K, _ROWS_PER_WORKER * _K)],
                    cs_v)

    zeros16 = jnp.zeros((_LANES,), jnp.float32)
    ones16 = jnp.ones((_LANES,), jnp.float32)

    def _zero_chunk(i, _):
        for c in range(8):
            buf[pl.ds((i * 8 + c) * _LANES, _LANES)] = zeros16
        return 0

    lax.fori_loop(0, _BATCH * _N // (8 * _LANES), _zero_chunk, 0)

    def _scatter_batch(b, values):
        for r8 in range(_BATCH):
            for c in range(_K // _LANES):
                cols = cs_v[pl.ds((b * _BATCH + r8) * _K + c * _LANES,
                                  _LANES)]
                plsc.store_scatter(buf, [cols + r8 * _N], values)

    for b in range(_ROWS_PER_WORKER // _BATCH):
        _scatter_batch(b, ones16)
        copy = pltpu.make_async_copy(
            buf,
            s_hbm.at[pl.ds((row_base + b * _BATCH) * _N, _BATCH * _N)],
            sem)
        copy.start()
        copy.wait()
        _scatter_batch(b, zeros16)


@jax.jit
def _sc_build_s(current_samples):
    mesh = plsc.VectorSubcoreMesh(core_axis_name="c", subcore_axis_name="s")
    s_flat = pl.kernel(
        _sc_build_s_body,
        out_type=jax.ShapeDtypeStruct((_N * _N,), jnp.float32),
        mesh=mesh,
        scratch_types=[
            pltpu.VMEM((_ROWS_PER_WORKER * _K,), jnp.int32),
            pltpu.VMEM((_BATCH * _N,), jnp.float32),
            pltpu.SemaphoreType.DMA,
        ],
        compiler_params=pltpu.CompilerParams(needs_layout_passes=False),
    )(current_samples.reshape(-1))
    return s_flat.reshape(_N, _N)


_BR = 256


def _tc_body(x_blk, xt_full, a_blk, s_full, s_own, cs_blk, out_ref):
    f32 = jnp.float32
    sims = lax.dot_general(
        x_blk[...], xt_full[...], (((1,), (0,)), ((), ())),
        preferred_element_type=f32)
    counts = lax.dot_general(
        a_blk[...], s_full[...], (((1,), (0,)), ((), ())),
        preferred_element_type=f32)
    cand = (counts > 0.0) | (s_own[...] > jnp.bfloat16(0.0))
    vals0 = jnp.where(cand, sims, f32(_MASKVAL))

    has_nbrs = jnp.max(a_blk[...], axis=1, keepdims=True) > jnp.bfloat16(0.0)

    iota_col = lax.broadcasted_iota(jnp.int32, (_BR, _N), 1)
    iota_k = lax.broadcasted_iota(jnp.int32, (_BR, _K), 1)

    def body(t, carry):
        vals, out = carry
        m = jnp.max(vals, axis=1, keepdims=True)
        eq = vals == m
        idx = jnp.min(jnp.where(eq, iota_col, _N), axis=1, keepdims=True)
        vals = jnp.where(iota_col == idx, -jnp.inf, vals)
        out = jnp.where(iota_k == t, idx, out)
        return vals, out

    out0 = jnp.zeros((_BR, _K), jnp.int32)
    _, top = lax.fori_loop(0, _K, body, (vals0, out0))

    out_ref[...] = jnp.where(has_nbrs, top, cs_blk[...])


@jax.jit
def _tc_topk(x, xt, a16, s16, current_samples):
    grid = (_N // _BR,)
    return pl.pallas_call(
        _tc_body,
        grid=grid,
        in_specs=[
            pl.BlockSpec((_BR, _D), lambda i: (i, 0)),
            pl.BlockSpec((_D, _N), lambda i: (0, 0)),
            pl.BlockSpec((_BR, _N), lambda i: (i, 0)),
            pl.BlockSpec((_N, _N), lambda i: (0, 0)),
            pl.BlockSpec((_BR, _N), lambda i: (i, 0)),
            pl.BlockSpec((_BR, _K), lambda i: (i, 0)),
        ],
        out_specs=pl.BlockSpec((_BR, _K), lambda i: (i, 0)),
        out_shape=jax.ShapeDtypeStruct((_N, _K), jnp.int32),
    )(x, xt, a16, s16, s16, current_samples)


def kernel(x, adj, current_samples):
    s = _sc_build_s(current_samples)
    s16 = s.astype(jnp.bfloat16)
    a16 = adj.astype(jnp.bfloat16)
    xt = x.T
    return _tc_topk(x, xt, a16, s16, current_samples)
